# baseline (speedup 1.0000x reference)
import jax
import jax.numpy as jnp
from jax import lax
from jax.experimental import pallas as pl
from jax.experimental.pallas import tpu as pltpu
from jax.experimental.pallas import tpu_sc as plsc

H = 384
W = 384
HALF = 256
D = 2 * HALF

NC = 2
NS = 16
NW = NC * NS
RPW = H // NW
JC = 48
NCHUNK = W // JC
NLANE = 16
NV = HALF // NLANE
NBUF = 4

_MESH = plsc.VectorSubcoreMesh(core_axis_name="c", subcore_axis_name="s")


def _sc_body(row_hbm, col_hbm, out_hbm, rowstage, *bufs_and_sems):
    bufs = bufs_and_sems[:NBUF]
    sems = bufs_and_sems[NBUF:]
    cid = lax.axis_index("c")
    sid = lax.axis_index("s")
    wid = sid * NC + cid
    base = wid * RPW
    off = pl.multiple_of(base * HALF, 8)
    pltpu.sync_copy(row_hbm.at[pl.ds(off, RPW * HALF)], rowstage)
    pend = [None] * NBUF
    for t in range(NCHUNK * RPW):
        c, i = divmod(t, RPW)
        j0 = c * JC
        k = t % NBUF
        buf = bufs[k]
        if pend[k] is not None:
            pend[k].wait()
        if i < NBUF:
            pltpu.sync_copy(
                col_hbm.at[pl.ds(j0, JC)], buf.at[:, pl.ds(HALF, HALF)]
            )
        regs = [
            rowstage[pl.ds(i * HALF + v * NLANE, NLANE)] for v in range(NV)
        ]

        def fill(j, carry, _buf=buf, _regs=regs):
            for v in range(NV):
                _buf[j, pl.ds(v * NLANE, NLANE)] = _regs[v]
            return carry

        lax.fori_loop(0, JC, fill, 0)
        pend[k] = pltpu.async_copy(
            buf, out_hbm.at[base + i, pl.ds(j0, JC)], sems[k]
        )
    for k in range(NBUF):
        if pend[k] is not None:
            pend[k].wait()


def kernel(row_embed, col_embed, h, w):
    del h, w
    run = pl.kernel(
        _sc_body,
        out_type=jax.ShapeDtypeStruct((H, W, D), jnp.float32),
        mesh=_MESH,
        scratch_types=(
            [pltpu.VMEM((RPW * HALF,), jnp.float32)]
            + [pltpu.VMEM((JC, D), jnp.float32)] * NBUF
            + [pltpu.SemaphoreType.DMA] * NBUF
        ),
    )
    return run(row_embed.reshape(-1), col_embed)


# speedup vs baseline: 1.6702x; 1.6702x over previous
import jax
import jax.numpy as jnp
from jax import lax
from jax.experimental import pallas as pl
from jax.experimental.pallas import tpu as pltpu
from jax.experimental.pallas import tpu_sc as plsc

H = 384
W = 384
HALF = 256
D = 2 * HALF

TCROWS = 256
SCROWS = H - TCROWS

NC = 2
NS = 16
NW = NC * NS
RPW = SCROWS // NW
JC = 96
NCHUNK = W // JC
NLANE = 16
NV = HALF // NLANE
NBUF = 2

BH = 8

_SC_MESH = plsc.VectorSubcoreMesh(core_axis_name="c", subcore_axis_name="s")


def _sc_body(row_hbm, col_hbm, out_hbm, rowstage, buf_a, buf_b, sem_a, sem_b):
    cid = lax.axis_index("c")
    sid = lax.axis_index("s")
    wid = sid * NC + cid
    base = TCROWS + wid * RPW
    off = pl.multiple_of(base * HALF, 8)
    pltpu.sync_copy(row_hbm.at[pl.ds(off, RPW * HALF)], rowstage)
    bufs = (buf_a, buf_b)
    sems = (sem_a, sem_b)
    pend = [None] * NBUF
    for t in range(NCHUNK * RPW):
        c, i = divmod(t, RPW)
        j0 = c * JC
        k = t % NBUF
        buf = bufs[k]
        if pend[k] is not None:
            pend[k].wait()
        if i < NBUF:
            pltpu.sync_copy(
                col_hbm.at[pl.ds(j0, JC)], buf.at[:, pl.ds(HALF, HALF)]
            )
        regs = [
            rowstage[pl.ds(i * HALF + v * NLANE, NLANE)] for v in range(NV)
        ]

        def fill(j, carry, _buf=buf, _regs=regs):
            for v in range(NV):
                _buf[j, pl.ds(v * NLANE, NLANE)] = _regs[v]
            return carry

        lax.fori_loop(0, JC, fill, 0)
        pend[k] = pltpu.async_copy(
            buf, out_hbm.at[base + i, pl.ds(j0, JC)], sems[k]
        )
    for k in range(NBUF):
        if pend[k] is not None:
            pend[k].wait()


def _tc_band_body(aliased_ref, row_ref, col_ref, out_ref):
    del aliased_ref
    row = row_ref[...]
    col = col_ref[...]
    out_ref[:, :, :HALF] = jnp.broadcast_to(row[:, None, :], (BH, W, HALF))
    out_ref[:, :, HALF:] = jnp.broadcast_to(col[None, :, :], (BH, W, HALF))


def kernel(row_embed, col_embed, h, w):
    del h, w
    sc_run = pl.kernel(
        _sc_body,
        out_type=jax.ShapeDtypeStruct((H, W, D), jnp.float32),
        mesh=_SC_MESH,
        scratch_types=[
            pltpu.VMEM((RPW * HALF,), jnp.float32),
            pltpu.VMEM((JC, D), jnp.float32),
            pltpu.VMEM((JC, D), jnp.float32),
            pltpu.SemaphoreType.DMA,
            pltpu.SemaphoreType.DMA,
        ],
    )
    sc_out = sc_run(row_embed.reshape(-1), col_embed)
    return pl.pallas_call(
        _tc_band_body,
        grid=(TCROWS // BH,),
        in_specs=[
            pl.BlockSpec(memory_space=pl.ANY),
            pl.BlockSpec((BH, HALF), lambda i: (i, 0)),
            pl.BlockSpec((W, HALF), lambda i: (0, 0)),
        ],
        out_specs=pl.BlockSpec((BH, W, D), lambda i: (i, 0, 0)),
        out_shape=jax.ShapeDtypeStruct((H, W, D), jnp.float32),
        input_output_aliases={0: 0},
    )(sc_out, row_embed[:TCROWS], col_embed[:W])


# speedup vs baseline: 2.1069x; 1.2615x over previous
import jax
import jax.numpy as jnp
from jax import lax
from jax.experimental import pallas as pl
from jax.experimental.pallas import tpu as pltpu
from jax.experimental.pallas import tpu_sc as plsc

H = 384
W = 384
HALF = 256
D = 2 * HALF

NC = 2
NS = 16
NW = NC * NS
NCG = 4
NRG = NW // NCG
RPW = H // NRG
JC = W // NCG
NLANE = 16
NV = HALF // NLANE

_MESH = plsc.VectorSubcoreMesh(core_axis_name="c", subcore_axis_name="s")


def _sc_body(row_hbm, col_hbm, out_hbm, rowstage, buf_a, buf_b, sem_a, sem_b):
    cid = lax.axis_index("c")
    sid = lax.axis_index("s")
    wid = sid * NC + cid
    rg = wid // NCG
    cc = wid % NCG
    row0 = rg * RPW
    j0 = cc * JC
    off = pl.multiple_of(row0 * HALF, 8)
    pltpu.sync_copy(row_hbm.at[pl.ds(off, RPW * HALF)], rowstage)
    bufs = (buf_a, buf_b)
    sems = (sem_a, sem_b)
    for k in (0, 1):
        pltpu.sync_copy(
            col_hbm.at[pl.ds(j0, JC)], bufs[k].at[:, pl.ds(HALF, HALF)]
        )
    pend = [None, None]
    for i in range(RPW):
        k = i % 2
        buf = bufs[k]
        if pend[k] is not None:
            pend[k].wait()
        regs = [
            rowstage[pl.ds(i * HALF + v * NLANE, NLANE)] for v in range(NV)
        ]

        def fill(j, carry, _buf=buf, _regs=regs):
            for v in range(NV):
                _buf[j, pl.ds(v * NLANE, NLANE)] = _regs[v]
            return carry

        lax.fori_loop(0, JC, fill, 0)
        pend[k] = pltpu.async_copy(
            buf, out_hbm.at[row0 + i, pl.ds(j0, JC)], sems[k]
        )
    for k in (0, 1):
        if pend[k] is not None:
            pend[k].wait()


def kernel(row_embed, col_embed, h, w):
    del h, w
    run = pl.kernel(
        _sc_body,
        out_type=jax.ShapeDtypeStruct((H, W, D), jnp.float32),
        mesh=_MESH,
        scratch_types=[
            pltpu.VMEM((RPW * HALF,), jnp.float32),
            pltpu.VMEM((JC, D), jnp.float32),
            pltpu.VMEM((JC, D), jnp.float32),
            pltpu.SemaphoreType.DMA,
            pltpu.SemaphoreType.DMA,
        ],
    )
    return run(row_embed.reshape(-1), col_embed)
